# baseline (speedup 1.0000x reference)
import functools

import jax
import jax.numpy as jnp
from jax.experimental import pallas as pl
from jax.experimental.pallas import tpu as pltpu
from jax.experimental.pallas import tpu_sc as plsc


def kernel(x, pos_embed):
    seq_len = x.shape[0]
    d_model = pos_embed.shape[1]
    mesh = plsc.VectorSubcoreMesh(
        core_axis_name="c", subcore_axis_name="s", num_cores=1, num_subcores=1
    )

    @functools.partial(
        pl.kernel,
        mesh=mesh,
        out_type=jax.ShapeDtypeStruct((seq_len, 1, d_model), pos_embed.dtype),
    )
    def sc_lookup(pe_hbm, out_hbm):
        pltpu.sync_copy(pe_hbm, out_hbm.at[:, 0, :])

    return sc_lookup(pos_embed[:seq_len])


# speedup vs baseline: 1.0942x; 1.0942x over previous
import functools

import jax
import jax.numpy as jnp
from jax.experimental import pallas as pl
from jax.experimental.pallas import tpu as pltpu
from jax.experimental.pallas import tpu_sc as plsc


def kernel(x, pos_embed):
    seq_len = x.shape[0]
    d_model = pos_embed.shape[1]
    mesh = plsc.ScalarSubcoreMesh(axis_name="c", num_cores=1)

    @functools.partial(
        pl.kernel,
        mesh=mesh,
        out_type=jax.ShapeDtypeStruct((seq_len, 1, d_model), pos_embed.dtype),
    )
    def sc_lookup(pe_hbm, out_hbm):
        pltpu.sync_copy(pe_hbm, out_hbm.at[:, 0, :])

    return sc_lookup(pos_embed[:seq_len])
